# baseline (speedup 1.0000x reference)
import functools

import jax
import jax.numpy as jnp
from jax import lax
from jax.experimental import pallas as pl
from jax.experimental.pallas import tpu as pltpu
from jax.experimental.pallas import tpu_sc as plsc

V = 1000
D = 1000
DP = 1024
T = 50
TP = 56
NC, NS = 2, 16
NW = NC * NS


@functools.partial(jax.jit, static_argnames=("B",))
def _gather_rows(table_p, idx_p, B):
    b_per_w = B // NW
    ipw = b_per_w * TP
    mesh = plsc.VectorSubcoreMesh(core_axis_name="c", subcore_axis_name="s")

    @functools.partial(
        pl.kernel,
        mesh=mesh,
        out_type=jax.ShapeDtypeStruct((B, TP, DP), jnp.float32),
        scratch_types=[
            pltpu.VMEM((ipw,), jnp.int32),
            pltpu.VMEM((TP, DP), jnp.float32),
            pltpu.VMEM((TP, DP), jnp.float32),
            pltpu.SemaphoreType.DMA,
            pltpu.SemaphoreType.DMA,
            pltpu.SemaphoreType.DMA,
            pltpu.SemaphoreType.DMA,
        ],
    )
    def k(table_hbm, idx_hbm, out_hbm, idx_v, buf0, buf1, g0, g1, w0, w1):
        wid = lax.axis_index("s") * NC + lax.axis_index("c")
        base = wid * b_per_w
        pltpu.sync_copy(idx_hbm.at[pl.ds(wid * ipw, ipw)], idx_v)

        def gather_start(j, buf, sem):
            pltpu.async_copy(table_hbm.at[idx_v.at[pl.ds(j * TP, TP)]], buf, sem)

        def gather_wait(j, buf, sem):
            pltpu.make_async_copy(
                table_hbm.at[idx_v.at[pl.ds(j * TP, TP)]], buf, sem
            ).wait()

        def write_start(j, buf, sem):
            pltpu.async_copy(buf, out_hbm.at[base + j], sem)

        def write_wait(j, buf, sem):
            pltpu.make_async_copy(buf, out_hbm.at[base + j], sem).wait()

        gather_start(0, buf0, g0)
        gather_start(1, buf1, g1)

        @pl.loop(0, b_per_w, step=2)
        def _(j):
            gather_wait(j, buf0, g0)
            write_start(j, buf0, w0)
            gather_wait(j + 1, buf1, g1)
            write_start(j + 1, buf1, w1)
            write_wait(j, buf0, w0)

            @pl.when(j + 2 < b_per_w)
            def _():
                gather_start(j + 2, buf0, g0)

            write_wait(j + 1, buf1, w1)

            @pl.when(j + 3 < b_per_w)
            def _():
                gather_start(j + 3, buf1, g1)

    return k(table_p, idx_p)


def kernel(idx, token_emb):
    B, T_ = idx.shape
    idx_p = jnp.pad(idx.astype(jnp.int32), ((0, 0), (0, TP - T))).reshape(-1)
    table_p = jnp.pad(token_emb, ((0, 0), (0, DP - D)))
    out = _gather_rows(table_p, idx_p, B)
    return out[:, :T, :D]
